# Initial kernel scaffold; baseline (speedup 1.0000x reference)
import jax
import jax.numpy as jnp
from jax.experimental import pallas as pl


def kernel(x, edge_index, W1, b1, W2, b2, W3, b3, Wfc, bfc):
    raise NotImplementedError("write your pallas kernel here")


# speedup vs baseline: 20.8171x; 20.8171x over previous
import jax
import jax.numpy as jnp
from jax import lax
from jax.experimental import pallas as pl
from jax.experimental.pallas import tpu as pltpu
from jax.experimental.pallas import tpu_sc as plsc

N_NODES = 50000
N_EDGES = 800000

NP = 50176
ROWS_PER_TILE = NP // 16
ROW_CHUNK = 784
EP = 802816
CHUNKS = EP // 512
CHUNKS_PER_TILE = 98
TRASH = N_NODES
BLK = 3136
GRID = NP // BLK


def _deg_body(dst_hbm, deg0_hbm, deg1_hbm, ones_v, idx_v, bounce_v, acc_sp):
    c = lax.axis_index("c")
    s = lax.axis_index("s")
    for k in range(8):
        ones_v[pl.ds(k * 16, 16)] = jnp.ones((16,), jnp.float32)
    for k in range(ROW_CHUNK // 16):
        bounce_v[pl.ds(k * 16, 16)] = jnp.zeros((16,), jnp.float32)
    base = s * ROWS_PER_TILE
    for k in range(4):
        pltpu.sync_copy(bounce_v, acc_sp.at[pl.ds(base + k * ROW_CHUNK, ROW_CHUNK)])
    plsc.subcore_barrier()

    def chunk_body(i, carry):
        cid = s * CHUNKS_PER_TILE + i * 2 + c
        pltpu.sync_copy(dst_hbm.at[cid], idx_v)
        for j in range(4):
            pltpu.sync_copy(ones_v, acc_sp.at[idx_v.at[j]], add=True)
        return carry

    lax.fori_loop(0, CHUNKS_PER_TILE // 2, chunk_body, 0)
    plsc.subcore_barrier()

    def writeback(deg_hbm):
        for k in range(4):
            sl = pl.ds(base + k * ROW_CHUNK, ROW_CHUNK)
            pltpu.sync_copy(acc_sp.at[sl], bounce_v)
            pltpu.sync_copy(bounce_v, deg_hbm.at[sl])

    @pl.when(c == 0)
    def _():
        writeback(deg0_hbm)

    @pl.when(c == 1)
    def _():
        writeback(deg1_hbm)


def _run_deg(dst3):
    mesh = plsc.VectorSubcoreMesh(core_axis_name="c", subcore_axis_name="s")
    f = pl.kernel(
        _deg_body,
        out_type=(jax.ShapeDtypeStruct((NP,), jnp.float32),
                  jax.ShapeDtypeStruct((NP,), jnp.float32)),
        mesh=mesh,
        scratch_types=[
            pltpu.VMEM((128,), jnp.float32),
            pltpu.VMEM((4, 128), jnp.int32),
            pltpu.VMEM((ROW_CHUNK,), jnp.float32),
            pltpu.VMEM_SHARED((NP,), jnp.float32),
        ],
        compiler_params=pltpu.CompilerParams(use_tc_tiling_on_sc=False),
    )
    return f(dst3)


def _make_agg_body(fh):
    def body(src_hbm, dst_hbm, y_lo, y_hi, acc_lo, acc_hi,
             sidx_v, didx_v, rows_v, acc_sp, gsem, ssem):
        c = lax.axis_index("c")
        s = lax.axis_index("s")
        base = s * ROWS_PER_TILE

        def linear(src_at, dst_at):
            for k in range(6):
                sl = pl.ds(base + k * 512, 512)
                pltpu.sync_copy(src_at(sl), rows_v)
                pltpu.sync_copy(rows_v, dst_at(sl))
            sl = pl.ds(base + 3072, 64)
            head = pl.ds(0, 64)
            pltpu.sync_copy(src_at(sl), rows_v.at[head])
            pltpu.sync_copy(rows_v.at[head], dst_at(sl))

        def run(y_ref, out_ref):
            linear(lambda sl: y_ref.at[sl], lambda sl: acc_sp.at[sl])
            plsc.subcore_barrier()

            def chunk_body(i, carry):
                cid = s * CHUNKS_PER_TILE + i
                pltpu.sync_copy(src_hbm.at[cid], sidx_v)
                pltpu.sync_copy(dst_hbm.at[cid], didx_v)
                gathers = [
                    pltpu.async_copy(y_ref.at[sidx_v.at[j]],
                                     rows_v.at[pl.ds(j * 128, 128)], gsem)
                    for j in range(4)
                ]
                for g in gathers:
                    g.wait()
                scatters = [
                    pltpu.async_copy(rows_v.at[pl.ds(j * 128, 128)],
                                     acc_sp.at[didx_v.at[j]], ssem, add=True)
                    for j in range(4)
                ]
                for t in scatters:
                    t.wait()
                return carry

            lax.fori_loop(0, CHUNKS_PER_TILE, chunk_body, 0)
            plsc.subcore_barrier()
            linear(lambda sl: acc_sp.at[sl], lambda sl: out_ref.at[sl])

        @pl.when(c == 0)
        def _():
            run(y_lo, acc_lo)

        @pl.when(c == 1)
        def _():
            run(y_hi, acc_hi)

    return body


def _run_agg(fh, src3, dst3, y_lo, y_hi):
    mesh = plsc.VectorSubcoreMesh(core_axis_name="c", subcore_axis_name="s")
    f = pl.kernel(
        _make_agg_body(fh),
        out_type=(jax.ShapeDtypeStruct((NP, fh), jnp.float32),
                  jax.ShapeDtypeStruct((NP, fh), jnp.float32)),
        mesh=mesh,
        scratch_types=[
            pltpu.VMEM((4, 128), jnp.int32),
            pltpu.VMEM((4, 128), jnp.int32),
            pltpu.VMEM((512, fh), jnp.float32),
            pltpu.VMEM_SHARED((NP, fh), jnp.float32),
            pltpu.SemaphoreType.DMA,
            pltpu.SemaphoreType.DMA,
        ],
        compiler_params=pltpu.CompilerParams(use_tc_tiling_on_sc=False),
    )
    return f(src3, dst3, y_lo, y_hi)


def _tc_stage0(deg_col, x_pad, w1):
    def body(deg_ref, x_ref, w_ref, dis_ref, ylo_ref, yhi_ref):
        deg = deg_ref[:, 0:1] + deg_ref[:, 1:2] + 1.0
        dis = lax.rsqrt(deg)
        dis_ref[...] = dis
        y = jnp.dot(x_ref[...], w_ref[...],
                    preferred_element_type=jnp.float32) * dis
        ylo_ref[...] = y[:, :16]
        yhi_ref[...] = y[:, 16:]

    return pl.pallas_call(
        body,
        grid=(GRID,),
        in_specs=[pl.BlockSpec((BLK, 2), lambda i: (i, 0)),
                  pl.BlockSpec((BLK, 2), lambda i: (i, 0)),
                  pl.BlockSpec((2, 32), lambda i: (0, 0))],
        out_specs=[pl.BlockSpec((BLK, 1), lambda i: (i, 0)),
                   pl.BlockSpec((BLK, 16), lambda i: (i, 0)),
                   pl.BlockSpec((BLK, 16), lambda i: (i, 0))],
        out_shape=[jax.ShapeDtypeStruct((NP, 1), jnp.float32),
                   jax.ShapeDtypeStruct((NP, 16), jnp.float32),
                   jax.ShapeDtypeStruct((NP, 16), jnp.float32)],
    )(deg_col, x_pad, w1)


def _tc_mid(acc_lo, acc_hi, dis, b, w):
    fh_in = acc_lo.shape[1]
    fin = 2 * fh_in
    fout = w.shape[1]
    fh_out = fout // 2

    def body(lo_ref, hi_ref, dis_ref, b_ref, w_ref, ylo_ref, yhi_ref):
        h = jnp.concatenate([lo_ref[...], hi_ref[...]], axis=1)
        dis = dis_ref[...]
        h = jnp.maximum(h * dis + b_ref[...], 0.0)
        y = jnp.dot(h, w_ref[...], preferred_element_type=jnp.float32) * dis
        ylo_ref[...] = y[:, :fh_out]
        yhi_ref[...] = y[:, fh_out:]

    return pl.pallas_call(
        body,
        grid=(GRID,),
        in_specs=[pl.BlockSpec((BLK, fh_in), lambda i: (i, 0)),
                  pl.BlockSpec((BLK, fh_in), lambda i: (i, 0)),
                  pl.BlockSpec((BLK, 1), lambda i: (i, 0)),
                  pl.BlockSpec((1, fin), lambda i: (0, 0)),
                  pl.BlockSpec((fin, fout), lambda i: (0, 0))],
        out_specs=[pl.BlockSpec((BLK, fh_out), lambda i: (i, 0)),
                   pl.BlockSpec((BLK, fh_out), lambda i: (i, 0))],
        out_shape=[jax.ShapeDtypeStruct((NP, fh_out), jnp.float32),
                   jax.ShapeDtypeStruct((NP, fh_out), jnp.float32)],
    )(acc_lo, acc_hi, dis, b, w)


def _tc_final(acc_lo, acc_hi, dis, b3, wfc, bfc):
    def body(lo_ref, hi_ref, dis_ref, b_ref, wfc_ref, bfc_ref, out_ref, colsum):
        i = pl.program_id(0)

        @pl.when(i == 0)
        def _():
            colsum[...] = jnp.zeros_like(colsum)

        h = jnp.concatenate([lo_ref[...], hi_ref[...]], axis=1)
        h = jnp.maximum(h * dis_ref[...] + b_ref[...], 0.0)
        row = i * BLK + lax.broadcasted_iota(jnp.int32, (BLK, 1), 0)
        h = jnp.where(row < N_NODES, h, 0.0)
        colsum[...] += jnp.sum(h, axis=0, keepdims=True)

        @pl.when(i == pl.num_programs(0) - 1)
        def _():
            m = colsum[...] * (1.0 / N_NODES)
            z = jnp.dot(m, wfc_ref[...],
                        preferred_element_type=jnp.float32) + bfc_ref[...]
            out_ref[...] = jax.nn.sigmoid(z)

    out = pl.pallas_call(
        body,
        grid=(GRID,),
        in_specs=[pl.BlockSpec((BLK, 32), lambda i: (i, 0)),
                  pl.BlockSpec((BLK, 32), lambda i: (i, 0)),
                  pl.BlockSpec((BLK, 1), lambda i: (i, 0)),
                  pl.BlockSpec((1, 64), lambda i: (0, 0)),
                  pl.BlockSpec((64, 1), lambda i: (0, 0)),
                  pl.BlockSpec((1, 1), lambda i: (0, 0))],
        out_specs=pl.BlockSpec((1, 1), lambda i: (0, 0)),
        out_shape=jax.ShapeDtypeStruct((1, 1), jnp.float32),
        scratch_shapes=[pltpu.VMEM((1, 64), jnp.float32)],
    )(acc_lo, acc_hi, dis, b3, wfc, bfc)
    return out.reshape(1)


def kernel(x, edge_index, W1, b1, W2, b2, W3, b3, Wfc, bfc):
    ei = edge_index.astype(jnp.int32)
    pad = jnp.full((EP - N_EDGES,), TRASH, jnp.int32)
    src3 = jnp.concatenate([ei[0], pad]).reshape(CHUNKS, 4, 128)
    dst3 = jnp.concatenate([ei[1], pad]).reshape(CHUNKS, 4, 128)
    x_pad = jnp.pad(x, ((0, NP - N_NODES), (0, 0)))

    deg0, deg1 = _run_deg(dst3)
    deg_col = jnp.stack([deg0, deg1], axis=1)
    dis, y_lo, y_hi = _tc_stage0(deg_col, x_pad, W1)
    a_lo, a_hi = _run_agg(16, src3, dst3, y_lo, y_hi)
    y_lo, y_hi = _tc_mid(a_lo, a_hi, dis, b1.reshape(1, -1), W2)
    a_lo, a_hi = _run_agg(32, src3, dst3, y_lo, y_hi)
    y_lo, y_hi = _tc_mid(a_lo, a_hi, dis, b2.reshape(1, -1), W3)
    a_lo, a_hi = _run_agg(32, src3, dst3, y_lo, y_hi)
    return _tc_final(a_lo, a_hi, dis, b3.reshape(1, -1), Wfc, bfc.reshape(1, 1))
